# baseline (speedup 1.0000x reference)
import functools

import jax
import jax.numpy as jnp
from jax import lax
from jax.experimental import pallas as pl
from jax.experimental.pallas import tpu as pltpu
from jax.experimental.pallas import tpu_sc as plsc

N_NODES = 1000000
D = 16
B = 16384
NC, NS = 2, 16
NW = NC * NS
BPW = B // NW
CHUNK = 128
NCHUNK = BPW // CHUNK


def _make_kernel():
    mesh = plsc.VectorSubcoreMesh(core_axis_name="c", subcore_axis_name="s")

    @functools.partial(
        pl.kernel,
        out_type=jax.ShapeDtypeStruct((B,), jnp.float32),
        mesh=mesh,
        compiler_params=pltpu.CompilerParams(
            needs_layout_passes=False, use_tc_tiling_on_sc=False),
        scratch_types=[
            pltpu.VMEM((NCHUNK, CHUNK), jnp.int32),
            pltpu.VMEM((NCHUNK, CHUNK), jnp.int32),
            pltpu.VMEM((BPW, D), jnp.float32),
            pltpu.VMEM((BPW, D), jnp.float32),
            pltpu.VMEM((BPW,), jnp.float32),
            pltpu.VMEM((16,), jnp.float32),
            pltpu.SemaphoreType.DMA,
        ],
    )
    def shallow_kernel(rx_hbm, tx_hbm, table_hbm, bias_hbm, out_hbm,
                       idx_rx, idx_tx, rows_rx, rows_tx, out_v, bias_v, sem):
        wid = lax.axis_index("s") * NC + lax.axis_index("c")

        pltpu.sync_copy(rx_hbm.at[pl.ds(wid * NCHUNK, NCHUNK)], idx_rx)
        pltpu.sync_copy(tx_hbm.at[pl.ds(wid * NCHUNK, NCHUNK)], idx_tx)
        pltpu.sync_copy(bias_hbm, bias_v)

        copies = []
        for c in range(NCHUNK):
            copies.append(pltpu.async_copy(
                table_hbm.at[idx_rx.at[c]],
                rows_rx.at[pl.ds(c * CHUNK, CHUNK)], sem))
            copies.append(pltpu.async_copy(
                table_hbm.at[idx_tx.at[c]],
                rows_tx.at[pl.ds(c * CHUNK, CHUNK)], sem))
        for cp in copies:
            cp.wait()

        lanes = lax.iota(jnp.int32, 16)
        bias_vec = bias_v[...]

        def group(g, _):
            row0 = g * 16
            ridx = row0 + lanes
            acc = jnp.zeros((16,), jnp.float32)
            for j in range(D):
                cidx = (lanes + j) & 15
                va = plsc.load_gather(rows_rx, [ridx, cidx])
                vb = plsc.load_gather(rows_tx, [ridx, cidx])
                acc = acc + va * vb
            z = acc + bias_vec
            out_v[pl.ds(row0, 16)] = 1.0 / (1.0 + jnp.exp(-z))
            return 0

        lax.fori_loop(0, BPW // 16, group, 0)

        pltpu.sync_copy(out_v, out_hbm.at[pl.ds(wid * BPW, BPW)])

    return shallow_kernel


_shallow = _make_kernel()


def kernel(rx, tx, emb_weight, bias):
    rx2 = rx.astype(jnp.int32).reshape(B // CHUNK, CHUNK)
    tx2 = tx.astype(jnp.int32).reshape(B // CHUNK, CHUNK)
    bias16 = jnp.broadcast_to(bias.astype(jnp.float32), (16,))
    return _shallow(rx2, tx2, emb_weight, bias16)


# speedup vs baseline: 3.1114x; 3.1114x over previous
import functools

import jax
import jax.numpy as jnp
from jax import lax
from jax.experimental import pallas as pl
from jax.experimental.pallas import tpu as pltpu
from jax.experimental.pallas import tpu_sc as plsc

D = 16
B = 16384
NC, NS = 2, 16
NW = NC * NS
BPW = B // NW
WAVE = 4
NWAVES = BPW // WAVE


def _make_kernel():
    mesh = plsc.VectorSubcoreMesh(core_axis_name="c", subcore_axis_name="s")

    @functools.partial(
        pl.kernel,
        out_type=jax.ShapeDtypeStruct((B,), jnp.float32),
        mesh=mesh,
        compiler_params=pltpu.CompilerParams(
            needs_layout_passes=False, use_tc_tiling_on_sc=True),
        scratch_types=[
            pltpu.VMEM((BPW,), jnp.int32),
            pltpu.VMEM((BPW,), jnp.int32),
            pltpu.VMEM((2, WAVE, D, 128), jnp.float32),
            pltpu.VMEM((2, WAVE, D, 128), jnp.float32),
            pltpu.VMEM((16 * D,), jnp.float32),
            pltpu.VMEM((BPW,), jnp.float32),
            pltpu.VMEM((16,), jnp.float32),
            pltpu.SemaphoreType.DMA,
            pltpu.SemaphoreType.DMA,
        ],
    )
    def shallow_kernel(rx_hbm, tx_hbm, tbl_hbm, bias_hbm, out_hbm,
                       idx_rx, idx_tx, slabs_a, slabs_b, pane, out_v,
                       bias_v, sem0, sem1):
        wid = lax.axis_index("s") * NC + lax.axis_index("c")
        base = wid * BPW

        pltpu.sync_copy(rx_hbm.at[pl.ds(base, BPW)], idx_rx)
        pltpu.sync_copy(tx_hbm.at[pl.ds(base, BPW)], idx_tx)
        pltpu.sync_copy(bias_hbm, bias_v)

        lanes = lax.iota(jnp.int32, 16)
        bias_vec = bias_v[...]

        def scalars_at(ref, b0):
            g16 = (b0 >> 4) << 4
            vec = ref[pl.ds(g16, 16)]
            lb = b0 & 15
            return [
                jnp.max(jnp.where(lanes == lb + l, vec, jnp.int32(-1)))
                for l in range(WAVE)
            ]

        def issue_wave(w, s, sem):
            b0 = w * WAVE
            irs = scalars_at(idx_rx, b0)
            its = scalars_at(idx_tx, b0)
            for l in range(WAVE):
                qr = pl.multiple_of((irs[l] >> 7) << 7, 128)
                qt = pl.multiple_of((its[l] >> 7) << 7, 128)
                pltpu.async_copy(
                    tbl_hbm.at[:, pl.ds(qr, 128)], slabs_a.at[s, l], sem)
                pltpu.async_copy(
                    tbl_hbm.at[:, pl.ds(qt, 128)], slabs_b.at[s, l], sem)

        def drain_wave(s, sem):
            for l in range(WAVE):
                pltpu.make_async_copy(
                    tbl_hbm.at[:, pl.ds(0, 128)], slabs_a.at[s, l], sem).wait()
                pltpu.make_async_copy(
                    tbl_hbm.at[:, pl.ds(0, 128)], slabs_b.at[s, l], sem).wait()

        def compute_wave(w, s):
            b0 = w * WAVE
            quarter = (w % 4) * WAVE
            sl = jnp.full((16,), s, jnp.int32)
            irs = scalars_at(idx_rx, b0)
            its = scalars_at(idx_tx, b0)
            for l in range(WAVE):
                r_r = jnp.full((16,), irs[l] & 127, jnp.int32)
                r_t = jnp.full((16,), its[l] & 127, jnp.int32)
                ll = jnp.full((16,), l, jnp.int32)
                va = plsc.load_gather(slabs_a, [sl, ll, lanes, r_r])
                vb = plsc.load_gather(slabs_b, [sl, ll, lanes, r_t])
                pane[pl.ds((quarter + l) * 16, 16)] = va * vb

        def reduce_pane(w):
            b0 = (w - 3) * WAVE
            acc = jnp.zeros((16,), jnp.float32)
            for j in range(D):
                cidx = ((lanes + j) & 15) + lanes * 16
                acc = acc + plsc.load_gather(pane, [cidx])
            z = acc + bias_vec
            out_v[pl.ds(b0, 16)] = 1.0 / (1.0 + jnp.exp(-z))

        issue_wave(0, 0, sem0)

        def step(t, _):
            w0 = 2 * t
            w1 = w0 + 1

            issue_wave(w1, 1, sem1)
            drain_wave(0, sem0)
            compute_wave(w0, 0)

            @pl.when(w0 + 2 < NWAVES)
            def _():
                issue_wave(w0 + 2, 0, sem0)

            drain_wave(1, sem1)
            compute_wave(w1, 1)

            @pl.when(t % 2 == 1)
            def _():
                reduce_pane(w1)

            return 0

        lax.fori_loop(0, NWAVES // 2, step, 0)

        pltpu.sync_copy(out_v, out_hbm.at[pl.ds(base, BPW)])

    return shallow_kernel


_shallow = _make_kernel()


def kernel(rx, tx, emb_weight, bias):
    bias16 = jnp.broadcast_to(bias.astype(jnp.float32), (16,))
    return _shallow(rx.astype(jnp.int32), tx.astype(jnp.int32),
                    emb_weight.T, bias16)


# speedup vs baseline: 3.7064x; 1.1912x over previous
import functools

import jax
import jax.numpy as jnp
from jax import lax
from jax.experimental import pallas as pl
from jax.experimental.pallas import tpu as pltpu
from jax.experimental.pallas import tpu_sc as plsc

D = 16
B = 16384
NC, NS = 2, 16
NW = NC * NS
BPW = B // NW
WAVE = 4
NWAVES = BPW // WAVE


def _make_kernel():
    mesh = plsc.VectorSubcoreMesh(core_axis_name="c", subcore_axis_name="s")

    @functools.partial(
        pl.kernel,
        out_type=jax.ShapeDtypeStruct((B,), jnp.float32),
        mesh=mesh,
        compiler_params=pltpu.CompilerParams(
            needs_layout_passes=False, use_tc_tiling_on_sc=True),
        scratch_types=[
            pltpu.VMEM((BPW,), jnp.int32),
            pltpu.VMEM((BPW,), jnp.int32),
            pltpu.VMEM((4, WAVE, D, 128), jnp.float32),
            pltpu.VMEM((4, WAVE, D, 128), jnp.float32),
            pltpu.VMEM((16 * D,), jnp.float32),
            pltpu.VMEM((BPW,), jnp.float32),
            pltpu.VMEM((16,), jnp.float32),
            pltpu.SemaphoreType.DMA,
            pltpu.SemaphoreType.DMA,
            pltpu.SemaphoreType.DMA,
            pltpu.SemaphoreType.DMA,
        ],
    )
    def shallow_kernel(rx_hbm, tx_hbm, tbl_hbm, bias_hbm, out_hbm,
                       idx_rx, idx_tx, slabs_a, slabs_b, pane, out_v,
                       bias_v, sem0, sem1, sem2, sem3):
        wid = lax.axis_index("s") * NC + lax.axis_index("c")
        base = wid * BPW

        pltpu.sync_copy(rx_hbm.at[pl.ds(base, BPW)], idx_rx)
        pltpu.sync_copy(tx_hbm.at[pl.ds(base, BPW)], idx_tx)
        pltpu.sync_copy(bias_hbm, bias_v)

        lanes = lax.iota(jnp.int32, 16)
        bias_vec = bias_v[...]

        def scalars_at(ref, b0):
            g16 = (b0 >> 4) << 4
            vec = ref[pl.ds(g16, 16)]
            lb = b0 & 15
            return [
                jnp.max(jnp.where(lanes == lb + l, vec, jnp.int32(-1)))
                for l in range(WAVE)
            ]

        def issue_wave(w, s, sem):
            b0 = w * WAVE
            irs = scalars_at(idx_rx, b0)
            its = scalars_at(idx_tx, b0)
            for l in range(WAVE):
                qr = pl.multiple_of((irs[l] >> 7) << 7, 128)
                qt = pl.multiple_of((its[l] >> 7) << 7, 128)
                pltpu.async_copy(
                    tbl_hbm.at[:, pl.ds(qr, 128)], slabs_a.at[s, l], sem)
                pltpu.async_copy(
                    tbl_hbm.at[:, pl.ds(qt, 128)], slabs_b.at[s, l], sem)

        def drain_wave(s, sem):
            for l in range(WAVE):
                pltpu.make_async_copy(
                    tbl_hbm.at[:, pl.ds(0, 128)], slabs_a.at[s, l], sem).wait()
                pltpu.make_async_copy(
                    tbl_hbm.at[:, pl.ds(0, 128)], slabs_b.at[s, l], sem).wait()

        def compute_wave(w, s):
            b0 = w * WAVE
            quarter = (w % 4) * WAVE
            sl = jnp.full((16,), s, jnp.int32)
            irs = scalars_at(idx_rx, b0)
            its = scalars_at(idx_tx, b0)
            for l in range(WAVE):
                r_r = jnp.full((16,), irs[l] & 127, jnp.int32)
                r_t = jnp.full((16,), its[l] & 127, jnp.int32)
                ll = jnp.full((16,), l, jnp.int32)
                va = plsc.load_gather(slabs_a, [sl, ll, lanes, r_r])
                vb = plsc.load_gather(slabs_b, [sl, ll, lanes, r_t])
                pane[pl.ds((quarter + l) * 16, 16)] = va * vb

        def reduce_pane(w):
            b0 = (w - 3) * WAVE
            acc = jnp.zeros((16,), jnp.float32)
            for j in range(D):
                cidx = ((lanes + j) & 15) + lanes * 16
                acc = acc + plsc.load_gather(pane, [cidx])
            z = acc + bias_vec
            out_v[pl.ds(b0, 16)] = 1.0 / (1.0 + jnp.exp(-z))

        sems = [sem0, sem1, sem2, sem3]
        issue_wave(0, 0, sem0)
        issue_wave(1, 1, sem1)
        issue_wave(2, 2, sem2)

        def step(t, _):
            w_base = 4 * t
            for k in range(4):
                w = w_base + k
                kn = (k + 3) % 4

                @pl.when(w + 3 < NWAVES)
                def _():
                    issue_wave(w + 3, kn, sems[kn])

                drain_wave(k, sems[k])
                compute_wave(w, k)

            reduce_pane(w_base + 3)
            return 0

        lax.fori_loop(0, NWAVES // 4, step, 0)

        pltpu.sync_copy(out_v, out_hbm.at[pl.ds(base, BPW)])

    return shallow_kernel


_shallow = _make_kernel()


def kernel(rx, tx, emb_weight, bias):
    bias16 = jnp.broadcast_to(bias.astype(jnp.float32), (16,))
    return _shallow(rx.astype(jnp.int32), tx.astype(jnp.int32),
                    emb_weight.T, bias16)
